# baseline (speedup 1.0000x reference)
import jax
import jax.numpy as jnp
from jax.experimental import pallas as pl
from jax.experimental.pallas import tpu as pltpu

_CH = 512
_NBUF = 5
_NOUT = 4


def _fused_router_kernel(
    x_hbm, w1_ref, b1_ref, w2t_ref, b2_ref, o_hbm, xbufs, xsems, sbufs, osems
):
    n_chunks = x_hbm.shape[0] // _CH
    w2 = w2t_ref[...].T

    def in_copy(c, slot):
        return pltpu.make_async_copy(
            x_hbm.at[pl.ds(c * _CH, _CH), :], xbufs.at[slot], xsems.at[slot]
        )

    def out_copy(c, slot):
        return pltpu.make_async_copy(
            sbufs.at[slot], o_hbm.at[:, pl.ds(c * _CH, _CH)], osems.at[slot]
        )

    for c in range(_NBUF):
        in_copy(c, c).start()

    def step(c, carry):
        slot = jax.lax.rem(c, _NBUF)
        oslot = jax.lax.rem(c, _NOUT)
        in_copy(c, slot).wait()
        xc = xbufs[slot]
        h = jnp.dot(xc, w1_ref[...], preferred_element_type=jnp.float32)
        h = jnp.maximum(h + b1_ref[...], 0.0)
        s = jnp.dot(h, w2, preferred_element_type=jnp.float32)

        @pl.when(c >= _NOUT)
        def _():
            out_copy(c - _NOUT, oslot).wait()

        sbufs[oslot] = (s + b2_ref[...]).T
        out_copy(c, oslot).start()

        @pl.when(c + _NBUF < n_chunks)
        def _():
            in_copy(c + _NBUF, slot).start()

        return carry

    jax.lax.fori_loop(0, n_chunks, step, 0)

    def drain(i, carry):
        c = n_chunks - _NOUT + i
        out_copy(c, jax.lax.rem(c, _NOUT)).wait()
        return carry

    jax.lax.fori_loop(0, _NOUT, drain, 0)


def kernel(x, W1, b1, W2, b2):
    M, K = x.shape
    H = W1.shape[1]
    G = W2.shape[1]

    out_t = pl.pallas_call(
        _fused_router_kernel,
        in_specs=[
            pl.BlockSpec(memory_space=pltpu.HBM),
            pl.BlockSpec(memory_space=pltpu.VMEM),
            pl.BlockSpec(memory_space=pltpu.VMEM),
            pl.BlockSpec(memory_space=pltpu.VMEM),
            pl.BlockSpec(memory_space=pltpu.VMEM),
        ],
        out_specs=pl.BlockSpec(memory_space=pltpu.HBM),
        out_shape=jax.ShapeDtypeStruct((G, M), jnp.float32),
        scratch_shapes=[
            pltpu.VMEM((_NBUF, _CH, K), jnp.float32),
            pltpu.SemaphoreType.DMA((_NBUF,)),
            pltpu.VMEM((_NOUT, G, _CH), jnp.float32),
            pltpu.SemaphoreType.DMA((_NOUT,)),
        ],
    )(x, W1, b1.reshape(1, H), W2.T, b2.reshape(1, G))
    return out_t.T


# speedup vs baseline: 1.0681x; 1.0681x over previous
import jax
import jax.numpy as jnp
from jax.experimental import pallas as pl
from jax.experimental.pallas import tpu as pltpu

_CH = 512
_NBUF = 5


def _fused_router_kernel(x_hbm, w1_ref, b1_ref, w2t_ref, b2_ref, o_ref, xbufs, sems):
    n_chunks = x_hbm.shape[0] // _CH
    w2 = w2t_ref[...].T

    def in_copy(c, slot):
        return pltpu.make_async_copy(
            x_hbm.at[pl.ds(c * _CH, _CH), :], xbufs.at[slot], sems.at[slot]
        )

    for c in range(_NBUF):
        in_copy(c, c).start()

    def step(c, carry):
        slot = jax.lax.rem(c, _NBUF)
        in_copy(c, slot).wait()
        xc = xbufs[slot]
        h = jnp.dot(xc, w1_ref[...], preferred_element_type=jnp.float32)
        h = jnp.maximum(h + b1_ref[...], 0.0)
        s = jnp.dot(h, w2, preferred_element_type=jnp.float32)
        o_ref[:, pl.ds(c * _CH, _CH)] = (s + b2_ref[...]).T

        @pl.when(c + _NBUF < n_chunks)
        def _():
            in_copy(c + _NBUF, slot).start()

        return carry

    jax.lax.fori_loop(0, n_chunks, step, 0)


def kernel(x, W1, b1, W2, b2):
    M, K = x.shape
    H = W1.shape[1]
    G = W2.shape[1]

    out_t = pl.pallas_call(
        _fused_router_kernel,
        in_specs=[
            pl.BlockSpec(memory_space=pltpu.HBM),
            pl.BlockSpec(memory_space=pltpu.VMEM),
            pl.BlockSpec(memory_space=pltpu.VMEM),
            pl.BlockSpec(memory_space=pltpu.VMEM),
            pl.BlockSpec(memory_space=pltpu.VMEM),
        ],
        out_specs=pl.BlockSpec(memory_space=pltpu.VMEM),
        out_shape=jax.ShapeDtypeStruct((G, M), jnp.float32),
        scratch_shapes=[
            pltpu.VMEM((_NBUF, _CH, K), jnp.float32),
            pltpu.SemaphoreType.DMA((_NBUF,)),
        ],
    )(x, W1, b1.reshape(1, H), W2.T, b2.reshape(1, G))
    return out_t.T


# speedup vs baseline: 1.1023x; 1.0320x over previous
import jax
import jax.numpy as jnp
from jax.experimental import pallas as pl
from jax.experimental.pallas import tpu as pltpu

_CH = 1024
_NBUF = 2


def _fused_router_kernel(x_hbm, w1_ref, b1_ref, w2t_ref, b2_ref, o_ref, xbufs, sems):
    n_chunks = x_hbm.shape[0] // _CH
    w2 = w2t_ref[...].T

    def in_copy(c, slot):
        return pltpu.make_async_copy(
            x_hbm.at[pl.ds(c * _CH, _CH), :], xbufs.at[slot], sems.at[slot]
        )

    for c in range(_NBUF):
        in_copy(c, c).start()

    def step(c, carry):
        slot = jax.lax.rem(c, _NBUF)
        in_copy(c, slot).wait()
        xc = xbufs[slot]
        h = jnp.dot(xc, w1_ref[...], preferred_element_type=jnp.float32)
        h = jnp.maximum(h + b1_ref[...], 0.0)
        s = jnp.dot(h, w2, preferred_element_type=jnp.float32)
        o_ref[:, pl.ds(c * _CH, _CH)] = (s + b2_ref[...]).T

        @pl.when(c + _NBUF < n_chunks)
        def _():
            in_copy(c + _NBUF, slot).start()

        return carry

    jax.lax.fori_loop(0, n_chunks, step, 0)


def kernel(x, W1, b1, W2, b2):
    M, K = x.shape
    H = W1.shape[1]
    G = W2.shape[1]

    out_t = pl.pallas_call(
        _fused_router_kernel,
        in_specs=[
            pl.BlockSpec(memory_space=pltpu.HBM),
            pl.BlockSpec(memory_space=pltpu.VMEM),
            pl.BlockSpec(memory_space=pltpu.VMEM),
            pl.BlockSpec(memory_space=pltpu.VMEM),
            pl.BlockSpec(memory_space=pltpu.VMEM),
        ],
        out_specs=pl.BlockSpec(memory_space=pltpu.VMEM),
        out_shape=jax.ShapeDtypeStruct((G, M), jnp.float32),
        scratch_shapes=[
            pltpu.VMEM((_NBUF, _CH, K), jnp.float32),
            pltpu.SemaphoreType.DMA((_NBUF,)),
        ],
    )(x, W1, b1.reshape(1, H), W2.T, b2.reshape(1, G))
    return out_t.T
